# baseline (speedup 1.0000x reference)
import functools
import math

import jax
import jax.numpy as jnp
from jax.experimental import pallas as pl
from jax.experimental.pallas import tpu as pltpu

NEG = -1e30


def _proj_body(x_ref, wq_ref, bq_ref, wk_ref, bk_ref, wv_ref, bv_ref,
               g1_ref, be1_ref, nq_ref, nk_ref, vf_ref):
    x = x_ref[...]
    f32 = jnp.float32
    dot = functools.partial(jax.lax.dot_general,
                            dimension_numbers=(((1,), (0,)), ((), ())),
                            preferred_element_type=f32)
    q = dot(x, wq_ref[...]) + bq_ref[...]
    k = dot(x, wk_ref[...]) + bk_ref[...]
    qn = jnp.sqrt(jnp.sum(q * q, axis=1, keepdims=True))
    kn = jnp.sqrt(jnp.sum(k * k, axis=1, keepdims=True))
    nq_ref[...] = q / jnp.maximum(qn, 1e-12)
    nk_ref[...] = k / jnp.maximum(kn, 1e-12)
    m = jnp.mean(x, axis=1, keepdims=True)
    xc = x - m
    v = jnp.mean(xc * xc, axis=1, keepdims=True)
    xln = xc * jax.lax.rsqrt(v + 1e-5) * g1_ref[...] + be1_ref[...]
    vf_ref[...] = dot(xln.astype(jnp.bfloat16),
                      wv_ref[...].astype(jnp.bfloat16)) + bv_ref[...]


def _attn_body(nq_ref, nk_ref, vf_ref, x_ref, wo_ref, bo_ref, rw_ref,
               g2_ref, be2_ref, wf1_ref, bf1_ref, wf2_ref, bf2_ref,
               out_ref, *, nk_count, heads):
    nq = nq_ref[0]
    nk = nk_ref[0]
    vf = vf_ref[0]
    dqk = nq.shape[1]
    d = vf.shape[1]
    hq = dqk // heads
    hv = d // heads
    dotT = functools.partial(jax.lax.dot_general,
                             dimension_numbers=(((1,), (1,)), ((), ())),
                             preferred_element_type=jnp.float32)
    dot = functools.partial(jax.lax.dot_general,
                            dimension_numbers=(((1,), (0,)), ((), ())),
                            preferred_element_type=jnp.float32)
    sim = dotT(nq, nk)
    work = sim
    for _ in range(nk_count - 1):
        mx = jnp.max(work, axis=1, keepdims=True)
        work = jnp.where(work == mx, NEG, work)
    thresh = jnp.max(work, axis=1, keepdims=True)
    bf = jnp.bfloat16
    maskb = jnp.where(sim >= thresh, 1.0, 0.0).astype(bf)
    scale = jnp.float32(1.0 / math.sqrt(hq))
    nqb = (nq * scale).astype(bf)
    nkb = nk.astype(bf)
    ones = jnp.ones((vf.shape[0], 1), jnp.float32)
    outs = []
    for h in range(heads):
        qh = nqb[:, h * hq:(h + 1) * hq]
        kh = nkb[:, h * hq:(h + 1) * hq]
        e = (jnp.exp(dotT(qh, kh)) * maskb).astype(bf)
        vh = jnp.concatenate([vf[:, h * hv:(h + 1) * hv], ones], axis=1)
        os_ = dot(e, vh.astype(bf))
        outs.append(os_[:, :hv] / os_[:, hv:hv + 1])
    sa = jnp.concatenate(outs, axis=1)
    sa = dot(sa.astype(bf), wo_ref[...].astype(bf)) + bo_ref[...]
    h1 = x_ref[0] + sa * rw_ref[...]
    m = jnp.mean(h1, axis=1, keepdims=True)
    hc = h1 - m
    v = jnp.mean(hc * hc, axis=1, keepdims=True)
    hln = hc * jax.lax.rsqrt(v + 1e-5) * g2_ref[...] + be2_ref[...]
    a = dot(hln.astype(bf), wf1_ref[...].astype(bf)) + bf1_ref[...]
    g = 0.5 * a * (1.0 + jax.lax.erf(a * jnp.float32(1.0 / math.sqrt(2.0))))
    ff = dot(g.astype(bf), wf2_ref[...].astype(bf)) + bf2_ref[...]
    out_ref[0] = h1 + ff * rw_ref[...]


def kernel(x, Wq, bq, Wk, bk, Wv, bv, Wo, bo, g1, be1, g2, be2, Wf1, bf1,
           Wf2, bf2, res_w):
    B, N, D = x.shape
    DQK = Wq.shape[1]
    DFF = Wf1.shape[1]
    H = 8
    NKN = 16
    BN = B * N
    f32 = jnp.float32

    x2 = x.reshape(BN, D)
    row = lambda a: a.reshape(1, -1)
    rw = res_w.reshape(1, 1)

    RA = 512
    nq2, nk2, vf2 = pl.pallas_call(
        _proj_body,
        grid=(BN // RA,),
        in_specs=[
            pl.BlockSpec((RA, D), lambda i: (i, 0)),
            pl.BlockSpec((D, DQK), lambda i: (0, 0)),
            pl.BlockSpec((1, DQK), lambda i: (0, 0)),
            pl.BlockSpec((D, DQK), lambda i: (0, 0)),
            pl.BlockSpec((1, DQK), lambda i: (0, 0)),
            pl.BlockSpec((D, D), lambda i: (0, 0)),
            pl.BlockSpec((1, D), lambda i: (0, 0)),
            pl.BlockSpec((1, D), lambda i: (0, 0)),
            pl.BlockSpec((1, D), lambda i: (0, 0)),
        ],
        out_specs=[
            pl.BlockSpec((RA, DQK), lambda i: (i, 0)),
            pl.BlockSpec((RA, DQK), lambda i: (i, 0)),
            pl.BlockSpec((RA, D), lambda i: (i, 0)),
        ],
        out_shape=[
            jax.ShapeDtypeStruct((BN, DQK), f32),
            jax.ShapeDtypeStruct((BN, DQK), f32),
            jax.ShapeDtypeStruct((BN, D), f32),
        ],
    )(x2, Wq, row(bq), Wk, row(bk), Wv, row(bv), row(g1), row(be1))

    nq3 = nq2.reshape(B, N, DQK)
    nk3 = nk2.reshape(B, N, DQK)
    vf3 = vf2.reshape(B, N, D)

    RC = 1024
    out = pl.pallas_call(
        functools.partial(_attn_body, nk_count=NKN, heads=H),
        grid=(B, N // RC),
        in_specs=[
            pl.BlockSpec((1, RC, DQK), lambda b, i: (b, i, 0)),
            pl.BlockSpec((1, N, DQK), lambda b, i: (b, 0, 0)),
            pl.BlockSpec((1, N, D), lambda b, i: (b, 0, 0)),
            pl.BlockSpec((1, RC, D), lambda b, i: (b, i, 0)),
            pl.BlockSpec((D, D), lambda b, i: (0, 0)),
            pl.BlockSpec((1, D), lambda b, i: (0, 0)),
            pl.BlockSpec((1, 1), lambda b, i: (0, 0)),
            pl.BlockSpec((1, D), lambda b, i: (0, 0)),
            pl.BlockSpec((1, D), lambda b, i: (0, 0)),
            pl.BlockSpec((D, DFF), lambda b, i: (0, 0)),
            pl.BlockSpec((1, DFF), lambda b, i: (0, 0)),
            pl.BlockSpec((DFF, D), lambda b, i: (0, 0)),
            pl.BlockSpec((1, D), lambda b, i: (0, 0)),
        ],
        out_specs=pl.BlockSpec((1, RC, D), lambda b, i: (b, i, 0)),
        out_shape=jax.ShapeDtypeStruct((B, N, D), f32),
    )(nq3, nk3, vf3, x, Wo, row(bo), rw, row(g2), row(be2), Wf1, row(bf1),
      Wf2, row(bf2))

    return out


# speedup vs baseline: 1.2838x; 1.2838x over previous
import functools
import math

import jax
import jax.numpy as jnp
from jax.experimental import pallas as pl
from jax.experimental.pallas import tpu as pltpu

NEG = -1e30


def _proj_body(x_ref, wq_ref, bq_ref, wk_ref, bk_ref, wv_ref, bv_ref,
               g1_ref, be1_ref, nq_ref, nk_ref, vf_ref):
    x = x_ref[...]
    f32 = jnp.float32
    dot = functools.partial(jax.lax.dot_general,
                            dimension_numbers=(((1,), (0,)), ((), ())),
                            preferred_element_type=f32)
    q = dot(x, wq_ref[...]) + bq_ref[...]
    k = dot(x, wk_ref[...]) + bk_ref[...]
    qn = jnp.sqrt(jnp.sum(q * q, axis=1, keepdims=True))
    kn = jnp.sqrt(jnp.sum(k * k, axis=1, keepdims=True))
    nq_ref[...] = q / jnp.maximum(qn, 1e-12)
    nk_ref[...] = k / jnp.maximum(kn, 1e-12)
    m = jnp.mean(x, axis=1, keepdims=True)
    xc = x - m
    v = jnp.mean(xc * xc, axis=1, keepdims=True)
    xln = xc * jax.lax.rsqrt(v + 1e-5) * g1_ref[...] + be1_ref[...]
    vf_ref[...] = dot(xln.astype(jnp.bfloat16),
                      wv_ref[...].astype(jnp.bfloat16)) + bv_ref[...]


def _attn_body(nq_ref, nk_ref, vf_ref, x_ref, wo_ref, bo_ref, rw_ref,
               g2_ref, be2_ref, wf1_ref, bf1_ref, wf2_ref, bf2_ref,
               out_ref, *, nk_count, heads):
    nq = nq_ref[0]
    nk = nk_ref[0]
    vf = vf_ref[0]
    dqk = nq.shape[1]
    d = vf.shape[1]
    hq = dqk // heads
    hv = d // heads
    dotT = functools.partial(jax.lax.dot_general,
                             dimension_numbers=(((1,), (1,)), ((), ())),
                             preferred_element_type=jnp.float32)
    dot = functools.partial(jax.lax.dot_general,
                            dimension_numbers=(((1,), (0,)), ((), ())),
                            preferred_element_type=jnp.float32)
    sim = dotT(nq, nk)
    work = sim
    for _ in range(nk_count // 2 - 1):
        m1 = jnp.max(work, axis=1, keepdims=True)
        m2 = jnp.max(jnp.where(work == m1, NEG, work), axis=1, keepdims=True)
        work = jnp.where(work >= m2, NEG, work)
    m1 = jnp.max(work, axis=1, keepdims=True)
    thresh = jnp.max(jnp.where(work == m1, NEG, work), axis=1, keepdims=True)
    bf = jnp.bfloat16
    maskb = jnp.where(sim >= thresh, 1.0, 0.0).astype(bf)
    scale = jnp.float32(1.0 / math.sqrt(hq))
    nqb = (nq * scale).astype(bf)
    nkb = nk.astype(bf)
    ones = jnp.ones((vf.shape[0], 1), jnp.float32)
    outs = []
    for h in range(heads):
        qh = nqb[:, h * hq:(h + 1) * hq]
        kh = nkb[:, h * hq:(h + 1) * hq]
        e = (jnp.exp(dotT(qh, kh)) * maskb).astype(bf)
        vh = jnp.concatenate([vf[:, h * hv:(h + 1) * hv], ones], axis=1)
        os_ = dot(e, vh.astype(bf))
        outs.append(os_[:, :hv] / os_[:, hv:hv + 1])
    sa = jnp.concatenate(outs, axis=1)
    sa = dot(sa.astype(bf), wo_ref[...].astype(bf)) + bo_ref[...]
    h1 = x_ref[0] + sa * rw_ref[...]
    m = jnp.mean(h1, axis=1, keepdims=True)
    hc = h1 - m
    v = jnp.mean(hc * hc, axis=1, keepdims=True)
    hln = hc * jax.lax.rsqrt(v + 1e-5) * g2_ref[...] + be2_ref[...]
    a = dot(hln.astype(bf), wf1_ref[...].astype(bf)) + bf1_ref[...]
    g = 0.5 * a * (1.0 + jax.lax.erf(a * jnp.float32(1.0 / math.sqrt(2.0))))
    ff = dot(g.astype(bf), wf2_ref[...].astype(bf)) + bf2_ref[...]
    out_ref[0] = h1 + ff * rw_ref[...]


def kernel(x, Wq, bq, Wk, bk, Wv, bv, Wo, bo, g1, be1, g2, be2, Wf1, bf1,
           Wf2, bf2, res_w):
    B, N, D = x.shape
    DQK = Wq.shape[1]
    DFF = Wf1.shape[1]
    H = 8
    NKN = 16
    BN = B * N
    f32 = jnp.float32

    x2 = x.reshape(BN, D)
    row = lambda a: a.reshape(1, -1)
    rw = res_w.reshape(1, 1)

    RA = 512
    nq2, nk2, vf2 = pl.pallas_call(
        _proj_body,
        grid=(BN // RA,),
        in_specs=[
            pl.BlockSpec((RA, D), lambda i: (i, 0)),
            pl.BlockSpec((D, DQK), lambda i: (0, 0)),
            pl.BlockSpec((1, DQK), lambda i: (0, 0)),
            pl.BlockSpec((D, DQK), lambda i: (0, 0)),
            pl.BlockSpec((1, DQK), lambda i: (0, 0)),
            pl.BlockSpec((D, D), lambda i: (0, 0)),
            pl.BlockSpec((1, D), lambda i: (0, 0)),
            pl.BlockSpec((1, D), lambda i: (0, 0)),
            pl.BlockSpec((1, D), lambda i: (0, 0)),
        ],
        out_specs=[
            pl.BlockSpec((RA, DQK), lambda i: (i, 0)),
            pl.BlockSpec((RA, DQK), lambda i: (i, 0)),
            pl.BlockSpec((RA, D), lambda i: (i, 0)),
        ],
        out_shape=[
            jax.ShapeDtypeStruct((BN, DQK), f32),
            jax.ShapeDtypeStruct((BN, DQK), f32),
            jax.ShapeDtypeStruct((BN, D), f32),
        ],
    )(x2, Wq, row(bq), Wk, row(bk), Wv, row(bv), row(g1), row(be1))

    nq3 = nq2.reshape(B, N, DQK)
    nk3 = nk2.reshape(B, N, DQK)
    vf3 = vf2.reshape(B, N, D)

    RC = 512
    out = pl.pallas_call(
        functools.partial(_attn_body, nk_count=NKN, heads=H),
        grid=(B, N // RC),
        in_specs=[
            pl.BlockSpec((1, RC, DQK), lambda b, i: (b, i, 0)),
            pl.BlockSpec((1, N, DQK), lambda b, i: (b, 0, 0)),
            pl.BlockSpec((1, N, D), lambda b, i: (b, 0, 0)),
            pl.BlockSpec((1, RC, D), lambda b, i: (b, i, 0)),
            pl.BlockSpec((D, D), lambda b, i: (0, 0)),
            pl.BlockSpec((1, D), lambda b, i: (0, 0)),
            pl.BlockSpec((1, 1), lambda b, i: (0, 0)),
            pl.BlockSpec((1, D), lambda b, i: (0, 0)),
            pl.BlockSpec((1, D), lambda b, i: (0, 0)),
            pl.BlockSpec((D, DFF), lambda b, i: (0, 0)),
            pl.BlockSpec((1, DFF), lambda b, i: (0, 0)),
            pl.BlockSpec((DFF, D), lambda b, i: (0, 0)),
            pl.BlockSpec((1, D), lambda b, i: (0, 0)),
        ],
        out_specs=pl.BlockSpec((1, RC, D), lambda b, i: (b, i, 0)),
        out_shape=jax.ShapeDtypeStruct((B, N, D), f32),
    )(nq3, nk3, vf3, x, Wo, row(bo), rw, row(g2), row(be2), Wf1, row(bf1),
      Wf2, row(bf2))

    return out


# speedup vs baseline: 1.2844x; 1.0005x over previous
import functools
import math

import jax
import jax.numpy as jnp
from jax.experimental import pallas as pl
from jax.experimental.pallas import tpu as pltpu

NEG = -1e30


def _proj_body(x_ref, wq_ref, bq_ref, wk_ref, bk_ref, wv_ref, bv_ref,
               g1_ref, be1_ref, nq_ref, nk_ref, vf_ref):
    x = x_ref[...]
    f32 = jnp.float32
    dot = functools.partial(jax.lax.dot_general,
                            dimension_numbers=(((1,), (0,)), ((), ())),
                            preferred_element_type=f32)
    q = dot(x, wq_ref[...]) + bq_ref[...]
    k = dot(x, wk_ref[...]) + bk_ref[...]
    qn = jnp.sqrt(jnp.sum(q * q, axis=1, keepdims=True))
    kn = jnp.sqrt(jnp.sum(k * k, axis=1, keepdims=True))
    nq_ref[...] = q / jnp.maximum(qn, 1e-12)
    nk_ref[...] = k / jnp.maximum(kn, 1e-12)
    m = jnp.mean(x, axis=1, keepdims=True)
    xc = x - m
    v = jnp.mean(xc * xc, axis=1, keepdims=True)
    xln = xc * jax.lax.rsqrt(v + 1e-5) * g1_ref[...] + be1_ref[...]
    vf_ref[...] = dot(xln.astype(jnp.bfloat16),
                      wv_ref[...].astype(jnp.bfloat16)) + bv_ref[...]


def _attn_body(nq_ref, nk_ref, vf_ref, x_ref, wo_ref, bo_ref, rw_ref,
               g2_ref, be2_ref, wf1_ref, bf1_ref, wf2_ref, bf2_ref,
               out_ref, *, nk_count, heads):
    nq = nq_ref[0]
    nk = nk_ref[0]
    vf = vf_ref[0]
    dqk = nq.shape[1]
    d = vf.shape[1]
    hq = dqk // heads
    hv = d // heads
    dotT = functools.partial(jax.lax.dot_general,
                             dimension_numbers=(((1,), (1,)), ((), ())),
                             preferred_element_type=jnp.float32)
    dot = functools.partial(jax.lax.dot_general,
                            dimension_numbers=(((1,), (0,)), ((), ())),
                            preferred_element_type=jnp.float32)
    sim = dotT(nq, nk)
    work = sim
    for _ in range(nk_count // 4 - 1):
        m1 = jnp.max(work, axis=1, keepdims=True)
        m2 = jnp.max(jnp.where(work == m1, NEG, work), axis=1, keepdims=True)
        m3 = jnp.max(jnp.where(work >= m2, NEG, work), axis=1, keepdims=True)
        m4 = jnp.max(jnp.where(work >= m3, NEG, work), axis=1, keepdims=True)
        work = jnp.where(work >= m4, NEG, work)
    m1 = jnp.max(work, axis=1, keepdims=True)
    m2 = jnp.max(jnp.where(work == m1, NEG, work), axis=1, keepdims=True)
    m3 = jnp.max(jnp.where(work >= m2, NEG, work), axis=1, keepdims=True)
    thresh = jnp.max(jnp.where(work >= m3, NEG, work), axis=1, keepdims=True)
    bf = jnp.bfloat16
    maskb = jnp.where(sim >= thresh, 1.0, 0.0).astype(bf)
    scale = jnp.float32(1.0 / math.sqrt(hq))
    nqb = (nq * scale).astype(bf)
    nkb = nk.astype(bf)
    ones = jnp.ones((vf.shape[0], 1), jnp.float32)
    outs = []
    for h in range(heads):
        qh = nqb[:, h * hq:(h + 1) * hq]
        kh = nkb[:, h * hq:(h + 1) * hq]
        e = (jnp.exp(dotT(qh, kh)) * maskb).astype(bf)
        vh = jnp.concatenate([vf[:, h * hv:(h + 1) * hv], ones], axis=1)
        os_ = dot(e, vh.astype(bf))
        outs.append(os_[:, :hv] / os_[:, hv:hv + 1])
    sa = jnp.concatenate(outs, axis=1)
    sa = dot(sa.astype(bf), wo_ref[...].astype(bf)) + bo_ref[...]
    h1 = x_ref[0] + sa * rw_ref[...]
    m = jnp.mean(h1, axis=1, keepdims=True)
    hc = h1 - m
    v = jnp.mean(hc * hc, axis=1, keepdims=True)
    hln = hc * jax.lax.rsqrt(v + 1e-5) * g2_ref[...] + be2_ref[...]
    a = dot(hln.astype(bf), wf1_ref[...].astype(bf)) + bf1_ref[...]
    g = 0.5 * a * (1.0 + jax.lax.erf(a * jnp.float32(1.0 / math.sqrt(2.0))))
    ff = dot(g.astype(bf), wf2_ref[...].astype(bf)) + bf2_ref[...]
    out_ref[0] = h1 + ff * rw_ref[...]


def kernel(x, Wq, bq, Wk, bk, Wv, bv, Wo, bo, g1, be1, g2, be2, Wf1, bf1,
           Wf2, bf2, res_w):
    B, N, D = x.shape
    DQK = Wq.shape[1]
    DFF = Wf1.shape[1]
    H = 8
    NKN = 16
    BN = B * N
    f32 = jnp.float32

    x2 = x.reshape(BN, D)
    row = lambda a: a.reshape(1, -1)
    rw = res_w.reshape(1, 1)

    RA = 512
    nq2, nk2, vf2 = pl.pallas_call(
        _proj_body,
        grid=(BN // RA,),
        in_specs=[
            pl.BlockSpec((RA, D), lambda i: (i, 0)),
            pl.BlockSpec((D, DQK), lambda i: (0, 0)),
            pl.BlockSpec((1, DQK), lambda i: (0, 0)),
            pl.BlockSpec((D, DQK), lambda i: (0, 0)),
            pl.BlockSpec((1, DQK), lambda i: (0, 0)),
            pl.BlockSpec((D, D), lambda i: (0, 0)),
            pl.BlockSpec((1, D), lambda i: (0, 0)),
            pl.BlockSpec((1, D), lambda i: (0, 0)),
            pl.BlockSpec((1, D), lambda i: (0, 0)),
        ],
        out_specs=[
            pl.BlockSpec((RA, DQK), lambda i: (i, 0)),
            pl.BlockSpec((RA, DQK), lambda i: (i, 0)),
            pl.BlockSpec((RA, D), lambda i: (i, 0)),
        ],
        out_shape=[
            jax.ShapeDtypeStruct((BN, DQK), f32),
            jax.ShapeDtypeStruct((BN, DQK), f32),
            jax.ShapeDtypeStruct((BN, D), f32),
        ],
    )(x2, Wq, row(bq), Wk, row(bk), Wv, row(bv), row(g1), row(be1))

    nq3 = nq2.reshape(B, N, DQK)
    nk3 = nk2.reshape(B, N, DQK)
    vf3 = vf2.reshape(B, N, D)

    RC = 512
    out = pl.pallas_call(
        functools.partial(_attn_body, nk_count=NKN, heads=H),
        grid=(B, N // RC),
        in_specs=[
            pl.BlockSpec((1, RC, DQK), lambda b, i: (b, i, 0)),
            pl.BlockSpec((1, N, DQK), lambda b, i: (b, 0, 0)),
            pl.BlockSpec((1, N, D), lambda b, i: (b, 0, 0)),
            pl.BlockSpec((1, RC, D), lambda b, i: (b, i, 0)),
            pl.BlockSpec((D, D), lambda b, i: (0, 0)),
            pl.BlockSpec((1, D), lambda b, i: (0, 0)),
            pl.BlockSpec((1, 1), lambda b, i: (0, 0)),
            pl.BlockSpec((1, D), lambda b, i: (0, 0)),
            pl.BlockSpec((1, D), lambda b, i: (0, 0)),
            pl.BlockSpec((D, DFF), lambda b, i: (0, 0)),
            pl.BlockSpec((1, DFF), lambda b, i: (0, 0)),
            pl.BlockSpec((DFF, D), lambda b, i: (0, 0)),
            pl.BlockSpec((1, D), lambda b, i: (0, 0)),
        ],
        out_specs=pl.BlockSpec((1, RC, D), lambda b, i: (b, i, 0)),
        out_shape=jax.ShapeDtypeStruct((B, N, D), f32),
    )(nq3, nk3, vf3, x, Wo, row(bo), rw, row(g2), row(be2), Wf1, row(bf1),
      Wf2, row(bf2))

    return out


# speedup vs baseline: 1.3436x; 1.0461x over previous
import functools
import math

import jax
import jax.numpy as jnp
from jax.experimental import pallas as pl
from jax.experimental.pallas import tpu as pltpu

NEG = -1e30


def _mega_body(x_ref, wq_ref, bq_ref, wk_ref, bk_ref, wv_ref, bv_ref,
               wo_ref, bo_ref, rw_ref, g1_ref, be1_ref, g2_ref, be2_ref,
               wf1_ref, bf1_ref, wf2_ref, bf2_ref, out_ref,
               nqs_ref, nks_ref, vfs_ref, *, nk_count, heads, rc):
    i = pl.program_id(1)
    f32 = jnp.float32
    bf = jnp.bfloat16
    dot = functools.partial(jax.lax.dot_general,
                            dimension_numbers=(((1,), (0,)), ((), ())),
                            preferred_element_type=f32)
    dotT = functools.partial(jax.lax.dot_general,
                             dimension_numbers=(((1,), (1,)), ((), ())),
                             preferred_element_type=f32)

    @pl.when(i == 0)
    def _proj():
        xb = x_ref[0]
        q = dot(xb, wq_ref[...]) + bq_ref[...]
        k = dot(xb, wk_ref[...]) + bk_ref[...]
        qn = jnp.sqrt(jnp.sum(q * q, axis=1, keepdims=True))
        kn = jnp.sqrt(jnp.sum(k * k, axis=1, keepdims=True))
        nqs_ref[...] = q / jnp.maximum(qn, 1e-12)
        nks_ref[...] = k / jnp.maximum(kn, 1e-12)
        m = jnp.mean(xb, axis=1, keepdims=True)
        xc = xb - m
        v = jnp.mean(xc * xc, axis=1, keepdims=True)
        xln = xc * jax.lax.rsqrt(v + 1e-5) * g1_ref[...] + be1_ref[...]
        vfs_ref[...] = dot(xln.astype(bf), wv_ref[...].astype(bf)) \
            + bv_ref[...]

    nq = nqs_ref[pl.ds(i * rc, rc), :]
    nk = nks_ref[...]
    vf = vfs_ref[...]
    xr = x_ref[0, pl.ds(i * rc, rc), :]
    dqk = nq.shape[1]
    d = vf.shape[1]
    hq = dqk // heads
    hv = d // heads

    sim = dotT(nq, nk)
    work = sim
    for _ in range(nk_count // 4 - 1):
        m1 = jnp.max(work, axis=1, keepdims=True)
        m2 = jnp.max(jnp.where(work == m1, NEG, work), axis=1, keepdims=True)
        m3 = jnp.max(jnp.where(work >= m2, NEG, work), axis=1, keepdims=True)
        m4 = jnp.max(jnp.where(work >= m3, NEG, work), axis=1, keepdims=True)
        work = jnp.where(work >= m4, NEG, work)
    m1 = jnp.max(work, axis=1, keepdims=True)
    m2 = jnp.max(jnp.where(work == m1, NEG, work), axis=1, keepdims=True)
    m3 = jnp.max(jnp.where(work >= m2, NEG, work), axis=1, keepdims=True)
    thresh = jnp.max(jnp.where(work >= m3, NEG, work), axis=1, keepdims=True)
    maskb = jnp.where(sim >= thresh, 1.0, 0.0).astype(bf)
    scale = jnp.float32(1.0 / math.sqrt(hq))
    nqb = (nq * scale).astype(bf)
    nkb = nk.astype(bf)
    ones = jnp.ones((vf.shape[0], 1), f32)
    outs = []
    for h in range(heads):
        qh = nqb[:, h * hq:(h + 1) * hq]
        kh = nkb[:, h * hq:(h + 1) * hq]
        e = (jnp.exp(dotT(qh, kh)) * maskb).astype(bf)
        vh = jnp.concatenate([vf[:, h * hv:(h + 1) * hv], ones], axis=1)
        os_ = dot(e, vh.astype(bf))
        outs.append(os_[:, :hv] / os_[:, hv:hv + 1])
    sa = jnp.concatenate(outs, axis=1)
    sa = dot(sa.astype(bf), wo_ref[...].astype(bf)) + bo_ref[...]
    h1 = xr + sa * rw_ref[...]
    m = jnp.mean(h1, axis=1, keepdims=True)
    hc = h1 - m
    v = jnp.mean(hc * hc, axis=1, keepdims=True)
    hln = hc * jax.lax.rsqrt(v + 1e-5) * g2_ref[...] + be2_ref[...]
    a = dot(hln.astype(bf), wf1_ref[...].astype(bf)) + bf1_ref[...]
    g = 0.5 * a * (1.0 + jax.lax.erf(a * jnp.float32(1.0 / math.sqrt(2.0))))
    ff = dot(g.astype(bf), wf2_ref[...].astype(bf)) + bf2_ref[...]
    out_ref[0] = h1 + ff * rw_ref[...]


def kernel(x, Wq, bq, Wk, bk, Wv, bv, Wo, bo, g1, be1, g2, be2, Wf1, bf1,
           Wf2, bf2, res_w):
    B, N, D = x.shape
    DQK = Wq.shape[1]
    DFF = Wf1.shape[1]
    H = 8
    NKN = 16
    f32 = jnp.float32

    row = lambda a: a.reshape(1, -1)
    rw = res_w.reshape(1, 1)

    RC = 512
    full = lambda shape: pl.BlockSpec(shape, lambda b, i: tuple(0 for _ in shape))
    out = pl.pallas_call(
        functools.partial(_mega_body, nk_count=NKN, heads=H, rc=RC),
        grid=(B, N // RC),
        in_specs=[
            pl.BlockSpec((1, N, D), lambda b, i: (b, 0, 0)),
            full((D, DQK)), full((1, DQK)),
            full((D, DQK)), full((1, DQK)),
            full((D, D)), full((1, D)),
            full((D, D)), full((1, D)),
            full((1, 1)),
            full((1, D)), full((1, D)),
            full((1, D)), full((1, D)),
            full((D, DFF)), full((1, DFF)),
            full((DFF, D)), full((1, D)),
        ],
        out_specs=pl.BlockSpec((1, RC, D), lambda b, i: (b, i, 0)),
        out_shape=jax.ShapeDtypeStruct((B, N, D), f32),
        scratch_shapes=[
            pltpu.VMEM((N, DQK), f32),
            pltpu.VMEM((N, DQK), f32),
            pltpu.VMEM((N, D), f32),
        ],
    )(x, Wq, row(bq), Wk, row(bk), Wv, row(bv), Wo, row(bo), rw,
      row(g1), row(be1), row(g2), row(be2), Wf1, row(bf1), Wf2, row(bf2))

    return out
